# baseline (speedup 1.0000x reference)
import functools

import jax
import jax.numpy as jnp
from jax import lax
from jax.experimental import pallas as pl
from jax.experimental.pallas import tpu as pltpu
from jax.experimental.pallas import tpu_sc as plsc

M = 100000
D = 64
B = 16384

NS = 16
L = 16
C = 12800
NCHUNKS = 8
TAIL = M - (NCHUNKS - 1) * C
RPT_FULL = C // NS
RPT_TAIL = (TAIL // NS) // 8 * 8
TAIL_REM = TAIL - NS * RPT_TAIL
TRASH = C
BPT = B // NS
IDXW = 128
NXFER = BPT // IDXW

_mesh = plsc.VectorSubcoreMesh(core_axis_name="c", subcore_axis_name="s")


@functools.partial(
    pl.kernel,
    out_type=jax.ShapeDtypeStruct((M, D), jnp.float32),
    mesh=_mesh,
    compiler_params=pltpu.CompilerParams(
        needs_layout_passes=False, use_tc_tiling_on_sc=False),
    scratch_types=[
        pltpu.VMEM((BPT,), jnp.int32),
        pltpu.VMEM((BPT,), jnp.int32),
        pltpu.VMEM((BPT,), jnp.int32),
        pltpu.VMEM((NXFER, IDXW), jnp.int32),
        pltpu.VMEM((NXFER, IDXW), jnp.int32),
        pltpu.VMEM((IDXW, D), jnp.float32),
        pltpu.VMEM_SHARED((C + NS, D), jnp.float32),
    ],
)
def _scatter_add_sc(mem_hbm, idx_hbm, val_hbm, out_hbm,
                    idx_v, pos1_v, lrow1_v, pos_v, lrow_v, valbuf, accum):
    c = lax.axis_index("c")
    s = lax.axis_index("s")
    my_trash = TRASH + s

    pltpu.sync_copy(idx_hbm.at[pl.ds(s * BPT, BPT)], idx_v)

    def copy_in(rpt, lo):
        pltpu.sync_copy(mem_hbm.at[pl.ds(lo + s * rpt, rpt)],
                        accum.at[pl.ds(s * rpt, rpt)])

    def copy_in_rem(lo):
        pltpu.sync_copy(mem_hbm.at[pl.ds(lo + NS * RPT_TAIL, TAIL_REM)],
                        accum.at[pl.ds(NS * RPT_TAIL, TAIL_REM)])

    def copy_out(rpt, lo):
        pltpu.sync_copy(accum.at[pl.ds(s * rpt, rpt)],
                        out_hbm.at[pl.ds(lo + s * rpt, rpt)])

    def copy_out_rem(lo):
        pltpu.sync_copy(accum.at[pl.ds(NS * RPT_TAIL, TAIL_REM)],
                        out_hbm.at[pl.ds(lo + NS * RPT_TAIL, TAIL_REM)])

    def copy_out_chunk(k):
        lo = (c + 2 * k) * C
        if k < NCHUNKS // 2 - 1:
            copy_out(RPT_FULL, lo)
        else:
            @pl.when(c == 0)
            def _():
                copy_out(RPT_FULL, lo)

            @pl.when(c == 1)
            def _():
                copy_out(RPT_TAIL, lo)

            @pl.when((c == 1) & (s == NS - 1))
            def _():
                copy_out_rem(lo)

    def copy_in_chunk(k):
        lo = (c + 2 * k) * C
        if k < NCHUNKS // 2 - 1:
            copy_in(RPT_FULL, lo)
        else:
            @pl.when(c == 0)
            def _():
                copy_in(RPT_FULL, lo)

            @pl.when(c == 1)
            def _():
                copy_in(RPT_TAIL, lo)

            @pl.when((c == 1) & (s == NS - 1))
            def _():
                copy_in_rem(lo)

    for k in range(NCHUNKS // 2):
        if k > 0:
            copy_out_chunk(k - 1)
        copy_in_chunk(k)
        plsc.subcore_barrier()

        lo = (c + 2 * k) * C
        hi = jnp.minimum(lo + C, M)

        zero16 = jnp.zeros((L,), jnp.int32)
        trash16 = jnp.full((L,), 0, jnp.int32) + my_trash
        for r in range(NXFER):
            for g in range(IDXW // L):
                pos_v[r, pl.ds(g * L, L)] = zero16
                lrow_v[r, pl.ds(g * L, L)] = trash16

        n = jnp.int32(0)
        lane = lax.iota(jnp.int32, L)
        ones = jnp.full((L,), 1, jnp.int32)
        zeros = jnp.zeros((L,), jnp.int32)
        for g in range(BPT // L):
            v = idx_v[pl.ds(g * L, L)]
            inr = (v >= lo) & (v < hi)
            inr_i = jnp.where(inr, ones, zeros)
            inc = plsc.cumsum(inr_i)
            dst = inc - inr_i + n
            plsc.store_scatter(pos1_v, [dst],
                               lane + (s * BPT + g * L), mask=inr)
            plsc.store_scatter(lrow1_v, [dst], v - lo, mask=inr)
            n = n + inc[L - 1]

        for r in range(NXFER):
            @pl.when(r * IDXW < n)
            def _(r=r):
                for g in range(IDXW // L):
                    off = r * IDXW + g * L
                    pv = pos1_v[pl.ds(off, L)]
                    lv = lrow1_v[pl.ds(off, L)]
                    covered = jnp.full((L,), off, jnp.int32) + lane < n
                    pos_v[r, pl.ds(g * L, L)] = jnp.where(covered, pv, zero16)
                    lrow_v[r, pl.ds(g * L, L)] = jnp.where(
                        covered, lv, trash16)

        for r in range(NXFER):
            @pl.when(r * IDXW < n)
            def _(r=r):
                pltpu.sync_copy(val_hbm.at[pos_v.at[r]], valbuf)
                pltpu.sync_copy(valbuf, accum.at[lrow_v.at[r]], add=True)

        plsc.subcore_barrier()

    copy_out_chunk(NCHUNKS // 2 - 1)


def kernel(mem, idx, val):
    return _scatter_add_sc(mem, idx.astype(jnp.int32), val)
